# Initial kernel scaffold; baseline (speedup 1.0000x reference)
import jax
import jax.numpy as jnp
from jax.experimental import pallas as pl


def kernel(encodings, embeddings):
    raise NotImplementedError("write your pallas kernel here")


# speedup vs baseline: 1.0955x; 1.0955x over previous
import jax
import jax.numpy as jnp
from jax import lax
from jax.experimental import pallas as pl
from jax.experimental.pallas import tpu as pltpu
from jax.experimental.pallas import tpu_sc as plsc

_D = 32
_B, _T = 16384, 50
_N = _B * _T
_NC, _NS = 2, 16
_NW = _NC * _NS
_PER_W = _N // _NW
_CHUNK = 1024
_NCH = _PER_W // _CHUNK


def _body(enc, table, out, idx_v, rows_v, sem):
    wid = lax.axis_index("s") * _NC + lax.axis_index("c")
    base = wid * _PER_W

    def step(j, carry):
        off = base + j * _CHUNK
        pltpu.sync_copy(enc.at[pl.ds(off, _CHUNK)], idx_v)
        pltpu.async_copy(table.at[idx_v], rows_v, sem).wait()
        pltpu.sync_copy(rows_v, out.at[pl.ds(off, _CHUNK)])
        return carry

    lax.fori_loop(0, _NCH, step, 0)


def kernel(encodings, embeddings):
    flat = encodings.reshape(_N)
    mesh = plsc.VectorSubcoreMesh(core_axis_name="c", subcore_axis_name="s")
    out = pl.kernel(
        _body,
        out_type=jax.ShapeDtypeStruct((_N, _D), jnp.float32),
        mesh=mesh,
        scratch_types=[
            pltpu.VMEM((_CHUNK,), jnp.int32),
            pltpu.VMEM((_CHUNK, _D), jnp.float32),
            pltpu.SemaphoreType.DMA,
        ],
        compiler_params=pltpu.CompilerParams(use_tc_tiling_on_sc=False),
    )(flat, embeddings)
    return out.reshape(_B, _T, _D)


# speedup vs baseline: 1.1141x; 1.0170x over previous
import jax
import jax.numpy as jnp
from jax import lax
from jax.experimental import pallas as pl
from jax.experimental.pallas import tpu as pltpu
from jax.experimental.pallas import tpu_sc as plsc

_D = 32
_B, _T = 16384, 50
_N = _B * _T
_NC, _NS = 2, 16
_NW = _NC * _NS
_PER_W = _N // _NW
_CHUNK = 640
_NCH = _PER_W // _CHUNK
_NBUF = 4
_AHEAD = 2


def _body(enc, table, out, idx_all, r0, r1, r2, r3,
          gs0, gs1, gs2, gs3, ws0, ws1, ws2, ws3):
    rows = [r0, r1, r2, r3]
    gsem = [gs0, gs1, gs2, gs3]
    wsem = [ws0, ws1, ws2, ws3]
    wid = lax.axis_index("s") * _NC + lax.axis_index("c")
    base = wid * _PER_W

    pltpu.sync_copy(enc.at[pl.ds(base, _PER_W)], idx_all)

    def gstart(j, b):
        pltpu.async_copy(
            table.at[idx_all.at[pl.ds(j * _CHUNK, _CHUNK)]], rows[b], gsem[b])

    def gwait(b):
        pltpu.make_async_copy(
            table.at[idx_all.at[pl.ds(0, _CHUNK)]], rows[b], gsem[b]).wait()

    def wstart(j, b):
        pltpu.async_copy(
            rows[b], out.at[pl.ds(base + j * _CHUNK, _CHUNK)], wsem[b])

    def wwait(b):
        pltpu.make_async_copy(
            rows[b], out.at[pl.ds(base, _CHUNK)], wsem[b]).wait()

    gstart(0, 0)
    gstart(1, 1)
    for j in range(_AHEAD):
        gwait(j % _NBUF)
        wstart(j, j % _NBUF)
        gstart(j + _AHEAD, (j + _AHEAD) % _NBUF)

    def outer(it, carry):
        j0 = _AHEAD + it * _NBUF
        for k in range(_NBUF):
            j = j0 + k
            b = (_AHEAD + k) % _NBUF
            bn = (_AHEAD + k + _AHEAD) % _NBUF
            gwait(b)
            wstart(j, b)
            wwait(bn)
            gstart(j + _AHEAD, bn)
        return carry

    lax.fori_loop(0, (_NCH - 2 * _AHEAD) // _NBUF, outer, 0)

    for j in range(_NCH - _AHEAD, _NCH):
        b = j % _NBUF
        gwait(b)
        wstart(j, b)

    for j in range(_NCH - _NBUF, _NCH):
        wwait(j % _NBUF)


def kernel(encodings, embeddings):
    flat = encodings.reshape(_N)
    mesh = plsc.VectorSubcoreMesh(core_axis_name="c", subcore_axis_name="s")
    out = pl.kernel(
        _body,
        out_type=jax.ShapeDtypeStruct((_N, _D), jnp.float32),
        mesh=mesh,
        scratch_types=(
            [pltpu.VMEM((_PER_W,), jnp.int32)]
            + [pltpu.VMEM((_CHUNK, _D), jnp.float32) for _ in range(_NBUF)]
            + [pltpu.SemaphoreType.DMA for _ in range(2 * _NBUF)]
        ),
        compiler_params=pltpu.CompilerParams(use_tc_tiling_on_sc=False),
    )(flat, embeddings)
    return out.reshape(_B, _T, _D)
